# Initial kernel scaffold; baseline (speedup 1.0000x reference)
import jax
import jax.numpy as jnp
from jax.experimental import pallas as pl


def kernel(dyn_feats, rel_feats, senders, receivers, frel_params, fdyn_params):
    raise NotImplementedError("write your pallas kernel here")


# speedup vs baseline: 29.4707x; 29.4707x over previous
import functools

import jax
import jax.numpy as jnp
from jax import lax
from jax.experimental import pallas as pl
from jax.experimental.pallas import tpu as pltpu
from jax.experimental.pallas import tpu_sc as plsc


_ETILE = 1024


def _edge_mlp_body(nreal_ref, rel, w1, b1, w2, b2, w3, b3, w4, b4, w5, b5, out):
    x = rel[0]
    x = jnp.maximum(jnp.dot(x, w1[...], preferred_element_type=jnp.float32) + b1[...], 0.0)
    x = jnp.maximum(jnp.dot(x, w2[...], preferred_element_type=jnp.float32) + b2[...], 0.0)
    x = jnp.maximum(jnp.dot(x, w3[...], preferred_element_type=jnp.float32) + b3[...], 0.0)
    x = jnp.maximum(jnp.dot(x, w4[...], preferred_element_type=jnp.float32) + b4[...], 0.0)
    x = jnp.dot(x, w5[...], preferred_element_type=jnp.float32) + b5[...]
    row = pl.program_id(1) * _ETILE + lax.broadcasted_iota(jnp.int32, x.shape, 0)
    out[0] = jnp.where(row < nreal_ref[0], x, 0.0)


def _full(shape):
    return pl.BlockSpec(shape, lambda b, i: (0, 0))


def _edge_mlp(rel_feats, frel_params, e_pad):
    B, E, D = rel_feats.shape
    ws = []
    in_specs = [pl.BlockSpec(memory_space=pltpu.SMEM),
                pl.BlockSpec((1, _ETILE, D), lambda b, i: (b, i, 0))]
    for (w, b) in frel_params:
        ws += [w, b.reshape(1, -1)]
        in_specs += [_full(w.shape), _full((1, b.shape[0]))]
    f = frel_params[-1][0].shape[1]
    return pl.pallas_call(
        _edge_mlp_body,
        grid=(B, e_pad // _ETILE),
        in_specs=in_specs,
        out_specs=pl.BlockSpec((1, _ETILE, f), lambda b, i: (b, i, 0)),
        out_shape=jax.ShapeDtypeStruct((B, e_pad, f), jnp.float32),
    )(jnp.array([E], jnp.int32), rel_feats, *ws)


def _node_mlp_body(dyn, agg, w1a, w1b, b1, w2, b2, w3, b3, w4, b4, w5, b5, out):
    x = (jnp.dot(dyn[...], w1a[...], preferred_element_type=jnp.float32)
         + jnp.dot(agg[...], w1b[...], preferred_element_type=jnp.float32)
         + b1[...])
    x = jnp.maximum(x, 0.0)
    x = jnp.maximum(jnp.dot(x, w2[...], preferred_element_type=jnp.float32) + b2[...], 0.0)
    x = jnp.maximum(jnp.dot(x, w3[...], preferred_element_type=jnp.float32) + b3[...], 0.0)
    x = jnp.maximum(jnp.dot(x, w4[...], preferred_element_type=jnp.float32) + b4[...], 0.0)
    out[...] = jnp.dot(x, w5[...], preferred_element_type=jnp.float32) + b5[...]


def _nfull(shape):
    return pl.BlockSpec(shape, lambda i: (0, 0))


def _node_mlp(dyn2, agg2, fdyn_params, tile):
    rows = dyn2.shape[0]
    assert rows % tile == 0
    d_dyn = dyn2.shape[1]
    (w1, b1) = fdyn_params[0]
    ws = [w1[:d_dyn], w1[d_dyn:], b1.reshape(1, -1)]
    in_specs = [
        pl.BlockSpec((tile, d_dyn), lambda i: (i, 0)),
        pl.BlockSpec((tile, agg2.shape[1]), lambda i: (i, 0)),
        _nfull(ws[0].shape), _nfull(ws[1].shape), _nfull((1, b1.shape[0])),
    ]
    for (w, b) in fdyn_params[1:]:
        ws += [w, b.reshape(1, -1)]
        in_specs += [_nfull(w.shape), _nfull((1, b.shape[0]))]
    d_out = fdyn_params[-1][0].shape[1]
    return pl.pallas_call(
        _node_mlp_body,
        grid=(rows // tile,),
        in_specs=in_specs,
        out_specs=pl.BlockSpec((tile, d_out), lambda i: (i, 0)),
        out_shape=jax.ShapeDtypeStruct((rows, d_out), jnp.float32),
    )(dyn2, agg2, *ws)


_CH = 128
_BPC = 16
_EB = _CH * _BPC


def _make_scatter(B, N, E_pad, F):
    NS = plsc.get_sparse_core_info().num_subcores
    NCH = E_pad // _CH
    NG = NCH // 8
    gper = NG // NS
    grem = NG - gper * NS
    NB = -(-(8 * (gper + 1)) // _BPC)
    mesh = plsc.VectorSubcoreMesh(core_axis_name="c", subcore_axis_name="s")

    @functools.partial(
        pl.kernel,
        out_type=jax.ShapeDtypeStruct((B * N, F), jnp.float32),
        mesh=mesh,
        scratch_types=[
            pltpu.VMEM((_BPC, _CH), jnp.int32),
            pltpu.VMEM((_EB, F), jnp.float32),
            pltpu.VMEM_SHARED((N, F), jnp.float32),
            pltpu.SemaphoreType.DMA,
        ],
        compiler_params=pltpu.CompilerParams(use_tc_tiling_on_sc=False),
    )
    def scatter_k(msg_hbm, recv_hbm, zeros_hbm, out_hbm, idx_v, msg_v, acc_sh, sem):
        b = lax.axis_index("c")
        s = lax.axis_index("s")

        @pl.when(s < NS - 1)
        def _():
            r0 = s * 640
            pltpu.sync_copy(zeros_hbm.at[pl.ds(r0, 640)], acc_sh.at[pl.ds(r0, 640)])

        @pl.when(s == NS - 1)
        def _():
            pltpu.sync_copy(zeros_hbm.at[pl.ds(9600, 400)], acc_sh.at[pl.ds(9600, 400)])

        plsc.subcore_barrier()

        base = 8 * (s * gper + jnp.minimum(s, grem))
        cnt = 8 * (gper + (s < grem).astype(jnp.int32))
        eoff = b * E_pad

        def blk_body(blk, carry):
            first = base + blk * _BPC
            start = jnp.minimum(first, base + cnt - _BPC)
            pltpu.sync_copy(recv_hbm.at[pl.ds(start, _BPC)], idx_v)
            pltpu.sync_copy(msg_hbm.at[pl.ds(eoff + start * _CH, _EB)], msg_v)
            for j in range(_BPC):
                @pl.when(start + j >= first)
                def _():
                    pltpu.async_copy(
                        msg_v.at[pl.ds(j * _CH, _CH)],
                        acc_sh.at[idx_v.at[j]], sem, add=True)
            for j in range(_BPC):
                @pl.when(start + j >= first)
                def _():
                    pltpu.make_async_copy(
                        msg_v.at[pl.ds(j * _CH, _CH)],
                        acc_sh.at[idx_v.at[j]], sem).wait()
            return carry

        lax.fori_loop(0, NB, blk_body, 0)
        plsc.subcore_barrier()

        @pl.when(s < NS - 1)
        def _():
            r0 = s * 640
            pltpu.sync_copy(acc_sh.at[pl.ds(r0, 640)],
                            out_hbm.at[pl.ds(b * N + r0, 640)])

        @pl.when(s == NS - 1)
        def _():
            pltpu.sync_copy(acc_sh.at[pl.ds(9600, 400)],
                            out_hbm.at[pl.ds(b * N + 9600, 400)])

    return scatter_k


def kernel(dyn_feats, rel_feats, senders, receivers, frel_params, fdyn_params):
    B, N, _ = dyn_feats.shape
    E = rel_feats.shape[1]
    F = frel_params[-1][0].shape[1]

    e_pad = -(-E // _ETILE) * _ETILE
    if (e_pad // _CH) % 8:
        e_pad = -(-(e_pad // _CH) // 8) * 8 * _CH

    msg = _edge_mlp(rel_feats, frel_params, e_pad)
    msg2 = msg.reshape(B * e_pad, F)

    recv_pad = jnp.zeros((e_pad,), jnp.int32).at[:E].set(receivers)
    recv2 = recv_pad.reshape(e_pad // _CH, _CH)
    zeros = jnp.zeros((N, F), jnp.float32)
    agg2 = _make_scatter(B, N, e_pad, F)(msg2, recv2, zeros)

    dyn2 = dyn_feats.reshape(B * N, dyn_feats.shape[-1])
    delta2 = _node_mlp(dyn2, agg2, fdyn_params, tile=5000)
    return delta2.reshape(B, N, fdyn_params[-1][0].shape[1])


# speedup vs baseline: 36.1898x; 1.2280x over previous
import functools

import jax
import jax.numpy as jnp
from jax import lax
from jax.experimental import pallas as pl
from jax.experimental.pallas import tpu as pltpu
from jax.experimental.pallas import tpu_sc as plsc


_ETILE = 1024


def _edge_mlp_body(nreal_ref, rel, w1, b1, w2, b2, w3, b3, w4, b4, w5, b5, out):
    x = rel[0]
    x = jnp.maximum(jnp.dot(x, w1[...], preferred_element_type=jnp.float32) + b1[...], 0.0)
    x = jnp.maximum(jnp.dot(x, w2[...], preferred_element_type=jnp.float32) + b2[...], 0.0)
    x = jnp.maximum(jnp.dot(x, w3[...], preferred_element_type=jnp.float32) + b3[...], 0.0)
    x = jnp.maximum(jnp.dot(x, w4[...], preferred_element_type=jnp.float32) + b4[...], 0.0)
    x = jnp.dot(x, w5[...], preferred_element_type=jnp.float32) + b5[...]
    row = pl.program_id(1) * _ETILE + lax.broadcasted_iota(jnp.int32, x.shape, 0)
    x = jnp.where(row < nreal_ref[0], x, 0.0)
    q = _ETILE // 8
    for k in range(8):
        out[0, :, 16 * k:16 * (k + 1)] = x[q * k:q * (k + 1), :]


def _full(shape):
    return pl.BlockSpec(shape, lambda b, i: (0, 0))


def _edge_mlp(rel_feats, frel_params, e_pad):
    B, E, D = rel_feats.shape
    ws = []
    in_specs = [pl.BlockSpec(memory_space=pltpu.SMEM),
                pl.BlockSpec((1, _ETILE, D), lambda b, i: (b, i, 0))]
    for (w, b) in frel_params:
        ws += [w, b.reshape(1, -1)]
        in_specs += [_full(w.shape), _full((1, b.shape[0]))]
    f = frel_params[-1][0].shape[1]
    assert f == 16
    return pl.pallas_call(
        _edge_mlp_body,
        grid=(B, e_pad // _ETILE),
        in_specs=in_specs,
        out_specs=pl.BlockSpec((1, _ETILE // 8, 128), lambda b, i: (b, i, 0)),
        out_shape=jax.ShapeDtypeStruct((B, e_pad // 8, 128), jnp.float32),
    )(jnp.array([E], jnp.int32), rel_feats, *ws)


def _node_mlp_body(dyn, agg, w1a, w1b, b1, w2, b2, w3, b3, w4, b4, w5, b5, out):
    x = (jnp.dot(dyn[...], w1a[...], preferred_element_type=jnp.float32)
         + jnp.dot(agg[...], w1b[...], preferred_element_type=jnp.float32)
         + b1[...])
    x = jnp.maximum(x, 0.0)
    x = jnp.maximum(jnp.dot(x, w2[...], preferred_element_type=jnp.float32) + b2[...], 0.0)
    x = jnp.maximum(jnp.dot(x, w3[...], preferred_element_type=jnp.float32) + b3[...], 0.0)
    x = jnp.maximum(jnp.dot(x, w4[...], preferred_element_type=jnp.float32) + b4[...], 0.0)
    out[...] = jnp.dot(x, w5[...], preferred_element_type=jnp.float32) + b5[...]


def _nfull(shape):
    return pl.BlockSpec(shape, lambda i: (0, 0))


def _node_mlp(dyn2, agg2, fdyn_params, tile):
    rows = dyn2.shape[0]
    assert rows % tile == 0
    d_dyn = dyn2.shape[1]
    (w1, b1) = fdyn_params[0]
    ws = [w1[:d_dyn], w1[d_dyn:], b1.reshape(1, -1)]
    in_specs = [
        pl.BlockSpec((tile, d_dyn), lambda i: (i, 0)),
        pl.BlockSpec((tile, agg2.shape[1]), lambda i: (i, 0)),
        _nfull(ws[0].shape), _nfull(ws[1].shape), _nfull((1, b1.shape[0])),
    ]
    for (w, b) in fdyn_params[1:]:
        ws += [w, b.reshape(1, -1)]
        in_specs += [_nfull(w.shape), _nfull((1, b.shape[0]))]
    d_out = fdyn_params[-1][0].shape[1]
    return pl.pallas_call(
        _node_mlp_body,
        grid=(rows // tile,),
        in_specs=in_specs,
        out_specs=pl.BlockSpec((tile, d_out), lambda i: (i, 0)),
        out_shape=jax.ShapeDtypeStruct((rows, d_out), jnp.float32),
    )(dyn2, agg2, *ws)


_CH = 128
_BPC = 16
_EB = _CH * _BPC


def _make_scatter(B, N, E_pad, F):
    NS = plsc.get_sparse_core_info().num_subcores
    NCH = E_pad // _CH
    NG = NCH // 8
    gper = NG // NS
    grem = NG - gper * NS
    NB = -(-(8 * (gper + 1)) // _BPC)
    mesh = plsc.VectorSubcoreMesh(core_axis_name="c", subcore_axis_name="s")

    @functools.partial(
        pl.kernel,
        out_type=jax.ShapeDtypeStruct((B * N, F), jnp.float32),
        mesh=mesh,
        scratch_types=[
            pltpu.VMEM((_BPC, _CH), jnp.int32),
            pltpu.VMEM((_EB, F), jnp.float32),
            pltpu.VMEM_SHARED((N, F), jnp.float32),
            pltpu.SemaphoreType.DMA,
        ],
        compiler_params=pltpu.CompilerParams(use_tc_tiling_on_sc=False),
    )
    def scatter_k(msg_hbm, recv_hbm, zeros_hbm, out_hbm, idx_v, msg_v, acc_sh, sem):
        b = lax.axis_index("c")
        s = lax.axis_index("s")

        @pl.when(s < NS - 1)
        def _():
            r0 = s * 640
            pltpu.sync_copy(zeros_hbm.at[pl.ds(r0, 640)], acc_sh.at[pl.ds(r0, 640)])

        @pl.when(s == NS - 1)
        def _():
            pltpu.sync_copy(zeros_hbm.at[pl.ds(9600, 400)], acc_sh.at[pl.ds(9600, 400)])

        plsc.subcore_barrier()

        base = 8 * (s * gper + jnp.minimum(s, grem))
        cnt = 8 * (gper + (s < grem).astype(jnp.int32))
        eoff = b * E_pad

        def blk_body(blk, carry):
            first = base + blk * _BPC
            start = jnp.minimum(first, base + cnt - _BPC)
            pltpu.sync_copy(recv_hbm.at[pl.ds(start, _BPC)], idx_v)
            pltpu.sync_copy(msg_hbm.at[pl.ds(eoff + start * _CH, _EB)], msg_v)
            for j in range(_BPC):
                @pl.when(start + j >= first)
                def _():
                    pltpu.async_copy(
                        msg_v.at[pl.ds(j * _CH, _CH)],
                        acc_sh.at[idx_v.at[j]], sem, add=True)
            for j in range(_BPC):
                @pl.when(start + j >= first)
                def _():
                    pltpu.make_async_copy(
                        msg_v.at[pl.ds(j * _CH, _CH)],
                        acc_sh.at[idx_v.at[j]], sem).wait()
            return carry

        lax.fori_loop(0, NB, blk_body, 0)
        plsc.subcore_barrier()

        @pl.when(s < NS - 1)
        def _():
            r0 = s * 640
            pltpu.sync_copy(acc_sh.at[pl.ds(r0, 640)],
                            out_hbm.at[pl.ds(b * N + r0, 640)])

        @pl.when(s == NS - 1)
        def _():
            pltpu.sync_copy(acc_sh.at[pl.ds(9600, 400)],
                            out_hbm.at[pl.ds(b * N + 9600, 400)])

    return scatter_k


def kernel(dyn_feats, rel_feats, senders, receivers, frel_params, fdyn_params):
    B, N, _ = dyn_feats.shape
    E = rel_feats.shape[1]
    F = frel_params[-1][0].shape[1]

    e_pad = -(-E // _ETILE) * _ETILE
    if (e_pad // _CH) % 8:
        e_pad = -(-(e_pad // _CH) // 8) * 8 * _CH

    msg = _edge_mlp(rel_feats, frel_params, e_pad)
    msg2 = msg.reshape(B * e_pad, F)

    recv_pad = jnp.zeros((e_pad,), jnp.int32).at[:E].set(receivers)
    recv_lin = recv_pad.reshape(-1, 8, _CH).transpose(0, 2, 1).reshape(-1)
    recv2 = recv_lin.reshape(e_pad // _CH, _CH)
    zeros = jnp.zeros((N, F), jnp.float32)
    agg2 = _make_scatter(B, N, e_pad, F)(msg2, recv2, zeros)

    dyn2 = dyn_feats.reshape(B * N, dyn_feats.shape[-1])
    delta2 = _node_mlp(dyn2, agg2, fdyn_params, tile=5000)
    return delta2.reshape(B, N, fdyn_params[-1][0].shape[1])


# speedup vs baseline: 48.9255x; 1.3519x over previous
import functools

import jax
import jax.numpy as jnp
from jax import lax
from jax.experimental import pallas as pl
from jax.experimental.pallas import tpu as pltpu
from jax.experimental.pallas import tpu_sc as plsc


_ETILE = 2048


def _edge_mlp_body(nreal_ref, rel, w1, b1, w2, b2, w3, b3, w4, b4, w5, b5, out):
    x = rel[0].astype(jnp.bfloat16)
    def lyr(x, w, b):
        y = jnp.dot(x, w[...], preferred_element_type=jnp.float32) + b[...]
        return jnp.maximum(y, 0.0).astype(jnp.bfloat16)
    x = lyr(x, w1, b1)
    x = lyr(x, w2, b2)
    x = lyr(x, w3, b3)
    x = lyr(x, w4, b4)
    x = jnp.dot(x, w5[...], preferred_element_type=jnp.float32) + b5[...]
    row = pl.program_id(1) * _ETILE + lax.broadcasted_iota(jnp.int32, x.shape, 0)
    x = jnp.where(row < nreal_ref[0], x, 0.0)
    q = _ETILE // 8
    for k in range(8):
        out[0, :, 16 * k:16 * (k + 1)] = x[q * k:q * (k + 1), :]


def _full(shape):
    return pl.BlockSpec(shape, lambda b, i: (0, 0))


def _edge_mlp(rel_feats, frel_params, e_pad):
    B, E, D = rel_feats.shape
    ws = []
    in_specs = [pl.BlockSpec(memory_space=pltpu.SMEM),
                pl.BlockSpec((1, _ETILE, D), lambda b, i: (b, i, 0))]
    for (w, b) in frel_params:
        ws += [w.astype(jnp.bfloat16), b.reshape(1, -1)]
        in_specs += [_full(w.shape), _full((1, b.shape[0]))]
    f = frel_params[-1][0].shape[1]
    assert f == 16
    return pl.pallas_call(
        _edge_mlp_body,
        grid=(B, e_pad // _ETILE),
        in_specs=in_specs,
        out_specs=pl.BlockSpec((1, _ETILE // 8, 128), lambda b, i: (b, i, 0)),
        out_shape=jax.ShapeDtypeStruct((B, e_pad // 8, 128), jnp.float32),
    )(jnp.array([E], jnp.int32), rel_feats, *ws)


def _node_mlp_body(dyn, agg, w1a, w1b, b1, w2, b2, w3, b3, w4, b4, w5, b5, out):
    x = (jnp.dot(dyn[...], w1a[...], preferred_element_type=jnp.float32)
         + jnp.dot(agg[...], w1b[...], preferred_element_type=jnp.float32)
         + b1[...])
    x = jnp.maximum(x, 0.0)
    x = jnp.maximum(jnp.dot(x, w2[...], preferred_element_type=jnp.float32) + b2[...], 0.0)
    x = jnp.maximum(jnp.dot(x, w3[...], preferred_element_type=jnp.float32) + b3[...], 0.0)
    x = jnp.maximum(jnp.dot(x, w4[...], preferred_element_type=jnp.float32) + b4[...], 0.0)
    out[...] = jnp.dot(x, w5[...], preferred_element_type=jnp.float32) + b5[...]


def _nfull(shape):
    return pl.BlockSpec(shape, lambda i: (0, 0))


def _node_mlp(dyn2, agg2, fdyn_params, tile):
    rows = dyn2.shape[0]
    assert rows % tile == 0
    d_dyn = dyn2.shape[1]
    (w1, b1) = fdyn_params[0]
    ws = [w1[:d_dyn], w1[d_dyn:], b1.reshape(1, -1)]
    in_specs = [
        pl.BlockSpec((tile, d_dyn), lambda i: (i, 0)),
        pl.BlockSpec((tile, agg2.shape[1]), lambda i: (i, 0)),
        _nfull(ws[0].shape), _nfull(ws[1].shape), _nfull((1, b1.shape[0])),
    ]
    for (w, b) in fdyn_params[1:]:
        ws += [w, b.reshape(1, -1)]
        in_specs += [_nfull(w.shape), _nfull((1, b.shape[0]))]
    d_out = fdyn_params[-1][0].shape[1]
    return pl.pallas_call(
        _node_mlp_body,
        grid=(rows // tile,),
        in_specs=in_specs,
        out_specs=pl.BlockSpec((tile, d_out), lambda i: (i, 0)),
        out_shape=jax.ShapeDtypeStruct((rows, d_out), jnp.float32),
    )(dyn2, agg2, *ws)


_CH = 128
_BPC = 16
_EB = _CH * _BPC


def _make_scatter(B, N, E_pad, F):
    NS = plsc.get_sparse_core_info().num_subcores
    NCH = E_pad // _CH
    NG = NCH // 8
    gper = NG // NS
    grem = NG - gper * NS
    NB = -(-(8 * (gper + 1)) // _BPC)
    mesh = plsc.VectorSubcoreMesh(core_axis_name="c", subcore_axis_name="s")

    @functools.partial(
        pl.kernel,
        out_type=jax.ShapeDtypeStruct((B * N, F), jnp.float32),
        mesh=mesh,
        scratch_types=[
            pltpu.VMEM((_BPC, _CH), jnp.int32),
            pltpu.VMEM((_EB, F), jnp.float32),
            pltpu.VMEM_SHARED((N, F), jnp.float32),
            pltpu.SemaphoreType.DMA,
        ],
        compiler_params=pltpu.CompilerParams(use_tc_tiling_on_sc=False),
    )
    def scatter_k(msg_hbm, recv_hbm, zeros_hbm, out_hbm, idx_v, msg_v, acc_sh, sem):
        b = lax.axis_index("c")
        s = lax.axis_index("s")

        @pl.when(s < NS - 1)
        def _():
            r0 = s * 640
            pltpu.sync_copy(zeros_hbm.at[pl.ds(r0, 640)], acc_sh.at[pl.ds(r0, 640)])

        @pl.when(s == NS - 1)
        def _():
            pltpu.sync_copy(zeros_hbm.at[pl.ds(9600, 400)], acc_sh.at[pl.ds(9600, 400)])

        plsc.subcore_barrier()

        base = 8 * (s * gper + jnp.minimum(s, grem))
        cnt = 8 * (gper + (s < grem).astype(jnp.int32))
        eoff = b * E_pad

        def blk_body(blk, carry):
            first = base + blk * _BPC
            start = jnp.minimum(first, base + cnt - _BPC)
            pltpu.sync_copy(recv_hbm.at[pl.ds(start, _BPC)], idx_v)
            pltpu.sync_copy(msg_hbm.at[pl.ds(eoff + start * _CH, _EB)], msg_v)
            for j in range(_BPC):
                @pl.when(start + j >= first)
                def _():
                    pltpu.async_copy(
                        msg_v.at[pl.ds(j * _CH, _CH)],
                        acc_sh.at[idx_v.at[j]], sem, add=True)
            for j in range(_BPC):
                @pl.when(start + j >= first)
                def _():
                    pltpu.make_async_copy(
                        msg_v.at[pl.ds(j * _CH, _CH)],
                        acc_sh.at[idx_v.at[j]], sem).wait()
            return carry

        lax.fori_loop(0, NB, blk_body, 0)
        plsc.subcore_barrier()

        @pl.when(s < NS - 1)
        def _():
            r0 = s * 640
            pltpu.sync_copy(acc_sh.at[pl.ds(r0, 640)],
                            out_hbm.at[pl.ds(b * N + r0, 640)])

        @pl.when(s == NS - 1)
        def _():
            pltpu.sync_copy(acc_sh.at[pl.ds(9600, 400)],
                            out_hbm.at[pl.ds(b * N + 9600, 400)])

    return scatter_k


def kernel(dyn_feats, rel_feats, senders, receivers, frel_params, fdyn_params):
    B, N, _ = dyn_feats.shape
    E = rel_feats.shape[1]
    F = frel_params[-1][0].shape[1]

    e_pad = -(-E // _ETILE) * _ETILE
    if (e_pad // _CH) % 8:
        e_pad = -(-(e_pad // _CH) // 8) * 8 * _CH

    msg = _edge_mlp(rel_feats, frel_params, e_pad)
    msg2 = msg.reshape(B * e_pad, F)

    recv_pad = jnp.zeros((e_pad,), jnp.int32).at[:E].set(receivers)
    recv_lin = recv_pad.reshape(-1, 8, _ETILE // 8).transpose(0, 2, 1).reshape(-1)
    recv2 = recv_lin.reshape(e_pad // _CH, _CH)
    zeros = jnp.zeros((N, F), jnp.float32)
    agg2 = _make_scatter(B, N, e_pad, F)(msg2, recv2, zeros)

    dyn2 = dyn_feats.reshape(B * N, dyn_feats.shape[-1])
    delta2 = _node_mlp(dyn2, agg2, fdyn_params, tile=5000)
    return delta2.reshape(B, N, fdyn_params[-1][0].shape[1])


# speedup vs baseline: 60.6055x; 1.2387x over previous
import functools

import jax
import jax.numpy as jnp
from jax import lax
from jax.experimental import pallas as pl
from jax.experimental.pallas import tpu as pltpu
from jax.experimental.pallas import tpu_sc as plsc


_ETILE = 2048


def _edge_mlp_body(nreal_ref, relt, w1, b1, w2, b2, w3, b3, w4, b4, w5, b5, out):
    xt = relt[...].astype(jnp.bfloat16)
    def lyr(x, w, b):
        y = jnp.dot(x, w[...], preferred_element_type=jnp.float32) + b[...]
        return jnp.maximum(y, 0.0).astype(jnp.bfloat16)
    x = lax.dot_general(xt, w1[...], (((0,), (0,)), ((), ())),
                        preferred_element_type=jnp.float32) + b1[...]
    x = jnp.maximum(x, 0.0).astype(jnp.bfloat16)
    x = lyr(x, w2, b2)
    x = lyr(x, w3, b3)
    x = lyr(x, w4, b4)
    x = jnp.dot(x, w5[...], preferred_element_type=jnp.float32) + b5[...]
    row = pl.program_id(1) * _ETILE + lax.broadcasted_iota(jnp.int32, x.shape, 0)
    x = jnp.where(row < nreal_ref[0], x, 0.0)
    q = _ETILE // 8
    for k in range(8):
        out[0, :, 16 * k:16 * (k + 1)] = x[q * k:q * (k + 1), :]


def _full(shape):
    return pl.BlockSpec(shape, lambda b, i: (0, 0))


def _edge_mlp(rel_feats, frel_params, e_pad):
    B, E, D = rel_feats.shape
    nb = e_pad // _ETILE
    relp = jnp.pad(rel_feats, ((0, 0), (0, e_pad - E), (0, 0)))
    relt = relp.transpose(2, 0, 1).reshape(D, B * e_pad)
    ws = []
    in_specs = [pl.BlockSpec(memory_space=pltpu.SMEM),
                pl.BlockSpec((D, _ETILE), lambda b, i: (0, b * nb + i))]
    for (w, b) in frel_params:
        ws += [w.astype(jnp.bfloat16), b.reshape(1, -1)]
        in_specs += [_full(w.shape), _full((1, b.shape[0]))]
    f = frel_params[-1][0].shape[1]
    assert f == 16
    return pl.pallas_call(
        _edge_mlp_body,
        grid=(B, nb),
        in_specs=in_specs,
        out_specs=pl.BlockSpec((1, _ETILE // 8, 128), lambda b, i: (b, i, 0)),
        out_shape=jax.ShapeDtypeStruct((B, e_pad // 8, 128), jnp.float32),
    )(jnp.array([E], jnp.int32), relt, *ws)


def _node_mlp_body(dyn, agg, w1a, w1b, b1, w2, b2, w3, b3, w4, b4, w5, b5, out):
    x = (jnp.dot(dyn[...], w1a[...], preferred_element_type=jnp.float32)
         + jnp.dot(agg[...], w1b[...], preferred_element_type=jnp.float32)
         + b1[...])
    x = jnp.maximum(x, 0.0)
    x = jnp.maximum(jnp.dot(x, w2[...], preferred_element_type=jnp.float32) + b2[...], 0.0)
    x = jnp.maximum(jnp.dot(x, w3[...], preferred_element_type=jnp.float32) + b3[...], 0.0)
    x = jnp.maximum(jnp.dot(x, w4[...], preferred_element_type=jnp.float32) + b4[...], 0.0)
    out[...] = jnp.dot(x, w5[...], preferred_element_type=jnp.float32) + b5[...]


def _nfull(shape):
    return pl.BlockSpec(shape, lambda i: (0, 0))


def _node_mlp(dyn2, agg2, fdyn_params, tile):
    rows = dyn2.shape[0]
    assert rows % tile == 0
    d_dyn = dyn2.shape[1]
    (w1, b1) = fdyn_params[0]
    ws = [w1[:d_dyn], w1[d_dyn:], b1.reshape(1, -1)]
    in_specs = [
        pl.BlockSpec((tile, d_dyn), lambda i: (i, 0)),
        pl.BlockSpec((tile, agg2.shape[1]), lambda i: (i, 0)),
        _nfull(ws[0].shape), _nfull(ws[1].shape), _nfull((1, b1.shape[0])),
    ]
    for (w, b) in fdyn_params[1:]:
        ws += [w, b.reshape(1, -1)]
        in_specs += [_nfull(w.shape), _nfull((1, b.shape[0]))]
    d_out = fdyn_params[-1][0].shape[1]
    return pl.pallas_call(
        _node_mlp_body,
        grid=(rows // tile,),
        in_specs=in_specs,
        out_specs=pl.BlockSpec((tile, d_out), lambda i: (i, 0)),
        out_shape=jax.ShapeDtypeStruct((rows, d_out), jnp.float32),
    )(dyn2, agg2, *ws)


_CH = 128
_BPC = 16
_EB = _CH * _BPC


def _make_scatter(B, N, E_pad, F):
    NS = plsc.get_sparse_core_info().num_subcores
    NCH = E_pad // _CH
    NG = NCH // 8
    gper = NG // NS
    grem = NG - gper * NS
    NB = -(-(8 * (gper + 1)) // _BPC)
    mesh = plsc.VectorSubcoreMesh(core_axis_name="c", subcore_axis_name="s")

    @functools.partial(
        pl.kernel,
        out_type=jax.ShapeDtypeStruct((B * N, F), jnp.float32),
        mesh=mesh,
        scratch_types=[
            pltpu.VMEM((_BPC, _CH), jnp.int32),
            pltpu.VMEM((_EB, F), jnp.float32),
            pltpu.VMEM_SHARED((N, F), jnp.float32),
            pltpu.SemaphoreType.DMA,
        ],
        compiler_params=pltpu.CompilerParams(use_tc_tiling_on_sc=False),
    )
    def scatter_k(msg_hbm, recv_hbm, zeros_hbm, out_hbm, idx_v, msg_v, acc_sh, sem):
        b = lax.axis_index("c")
        s = lax.axis_index("s")

        @pl.when(s < NS - 1)
        def _():
            r0 = s * 640
            pltpu.sync_copy(zeros_hbm.at[pl.ds(r0, 640)], acc_sh.at[pl.ds(r0, 640)])

        @pl.when(s == NS - 1)
        def _():
            pltpu.sync_copy(zeros_hbm.at[pl.ds(9600, 400)], acc_sh.at[pl.ds(9600, 400)])

        plsc.subcore_barrier()

        base = 8 * (s * gper + jnp.minimum(s, grem))
        cnt = 8 * (gper + (s < grem).astype(jnp.int32))
        eoff = b * E_pad

        def blk_body(blk, carry):
            first = base + blk * _BPC
            start = jnp.minimum(first, base + cnt - _BPC)
            pltpu.sync_copy(recv_hbm.at[pl.ds(start, _BPC)], idx_v)
            pltpu.sync_copy(msg_hbm.at[pl.ds(eoff + start * _CH, _EB)], msg_v)
            for j in range(_BPC):
                @pl.when(start + j >= first)
                def _():
                    pltpu.async_copy(
                        msg_v.at[pl.ds(j * _CH, _CH)],
                        acc_sh.at[idx_v.at[j]], sem, add=True)
            for j in range(_BPC):
                @pl.when(start + j >= first)
                def _():
                    pltpu.make_async_copy(
                        msg_v.at[pl.ds(j * _CH, _CH)],
                        acc_sh.at[idx_v.at[j]], sem).wait()
            return carry

        lax.fori_loop(0, NB, blk_body, 0)
        plsc.subcore_barrier()

        @pl.when(s < NS - 1)
        def _():
            r0 = s * 640
            pltpu.sync_copy(acc_sh.at[pl.ds(r0, 640)],
                            out_hbm.at[pl.ds(b * N + r0, 640)])

        @pl.when(s == NS - 1)
        def _():
            pltpu.sync_copy(acc_sh.at[pl.ds(9600, 400)],
                            out_hbm.at[pl.ds(b * N + 9600, 400)])

    return scatter_k


def kernel(dyn_feats, rel_feats, senders, receivers, frel_params, fdyn_params):
    B, N, _ = dyn_feats.shape
    E = rel_feats.shape[1]
    F = frel_params[-1][0].shape[1]

    e_pad = -(-E // _ETILE) * _ETILE
    if (e_pad // _CH) % 8:
        e_pad = -(-(e_pad // _CH) // 8) * 8 * _CH

    msg = _edge_mlp(rel_feats, frel_params, e_pad)
    msg2 = msg.reshape(B * e_pad, F)

    recv_pad = jnp.zeros((e_pad,), jnp.int32).at[:E].set(receivers)
    recv_lin = recv_pad.reshape(-1, 8, _ETILE // 8).transpose(0, 2, 1).reshape(-1)
    recv2 = recv_lin.reshape(e_pad // _CH, _CH)
    zeros = jnp.zeros((N, F), jnp.float32)
    agg2 = _make_scatter(B, N, e_pad, F)(msg2, recv2, zeros)

    dyn2 = dyn_feats.reshape(B * N, dyn_feats.shape[-1])
    delta2 = _node_mlp(dyn2, agg2, fdyn_params, tile=5000)
    return delta2.reshape(B, N, fdyn_params[-1][0].shape[1])


# speedup vs baseline: 73.2138x; 1.2080x over previous
import functools

import jax
import jax.numpy as jnp
from jax import lax
from jax.experimental import pallas as pl
from jax.experimental.pallas import tpu as pltpu
from jax.experimental.pallas import tpu_sc as plsc


_ETILE = 8192


def _edge_mlp_body(nreal_ref, relt, w1, b1, w2, b2, w3, b3, w4, b4, w5, b5, out):
    xt = relt[...].astype(jnp.bfloat16)
    def lyr(x, w, b):
        y = jnp.dot(x, w[...], preferred_element_type=jnp.float32) + b[...]
        return jnp.maximum(y, 0.0).astype(jnp.bfloat16)
    x = lax.dot_general(xt, w1[...], (((0,), (0,)), ((), ())),
                        preferred_element_type=jnp.float32) + b1[...]
    x = jnp.maximum(x, 0.0).astype(jnp.bfloat16)
    x = lyr(x, w2, b2)
    x = lyr(x, w3, b3)
    x = lyr(x, w4, b4)
    x = jnp.dot(x, w5[...], preferred_element_type=jnp.float32) + b5[...]
    row = pl.program_id(1) * _ETILE + lax.broadcasted_iota(jnp.int32, x.shape, 0)
    x = jnp.where(row < nreal_ref[0], x, 0.0)
    q = _ETILE // 8
    for k in range(8):
        out[0, :, 16 * k:16 * (k + 1)] = x[q * k:q * (k + 1), :]


def _full(shape):
    return pl.BlockSpec(shape, lambda b, i: (0, 0))


def _edge_mlp(rel_feats, frel_params, e_pad):
    B, E, D = rel_feats.shape
    nb = e_pad // _ETILE
    relp = jnp.pad(rel_feats, ((0, 0), (0, e_pad - E), (0, 0)))
    relt = relp.transpose(2, 0, 1).reshape(D, B * e_pad)
    ws = []
    in_specs = [pl.BlockSpec(memory_space=pltpu.SMEM),
                pl.BlockSpec((D, _ETILE), lambda b, i: (0, b * nb + i))]
    for (w, b) in frel_params:
        ws += [w.astype(jnp.bfloat16), b.reshape(1, -1)]
        in_specs += [_full(w.shape), _full((1, b.shape[0]))]
    f = frel_params[-1][0].shape[1]
    assert f == 16
    return pl.pallas_call(
        _edge_mlp_body,
        grid=(B, nb),
        in_specs=in_specs,
        out_specs=pl.BlockSpec((1, _ETILE // 8, 128), lambda b, i: (b, i, 0)),
        out_shape=jax.ShapeDtypeStruct((B, e_pad // 8, 128), jnp.float32),
    )(jnp.array([E], jnp.int32), relt, *ws)


def _node_mlp_body(dyn, agg, w1a, w1b, b1, w2, b2, w3, b3, w4, b4, w5, b5, out):
    x = (jnp.dot(dyn[...], w1a[...], preferred_element_type=jnp.float32)
         + jnp.dot(agg[...], w1b[...], preferred_element_type=jnp.float32)
         + b1[...])
    x = jnp.maximum(x, 0.0)
    x = jnp.maximum(jnp.dot(x, w2[...], preferred_element_type=jnp.float32) + b2[...], 0.0)
    x = jnp.maximum(jnp.dot(x, w3[...], preferred_element_type=jnp.float32) + b3[...], 0.0)
    x = jnp.maximum(jnp.dot(x, w4[...], preferred_element_type=jnp.float32) + b4[...], 0.0)
    out[...] = jnp.dot(x, w5[...], preferred_element_type=jnp.float32) + b5[...]


def _nfull(shape):
    return pl.BlockSpec(shape, lambda i: (0, 0))


def _node_mlp(dyn2, agg2, fdyn_params, tile):
    rows = dyn2.shape[0]
    assert rows % tile == 0
    d_dyn = dyn2.shape[1]
    (w1, b1) = fdyn_params[0]
    ws = [w1[:d_dyn], w1[d_dyn:], b1.reshape(1, -1)]
    in_specs = [
        pl.BlockSpec((tile, d_dyn), lambda i: (i, 0)),
        pl.BlockSpec((tile, agg2.shape[1]), lambda i: (i, 0)),
        _nfull(ws[0].shape), _nfull(ws[1].shape), _nfull((1, b1.shape[0])),
    ]
    for (w, b) in fdyn_params[1:]:
        ws += [w, b.reshape(1, -1)]
        in_specs += [_nfull(w.shape), _nfull((1, b.shape[0]))]
    d_out = fdyn_params[-1][0].shape[1]
    return pl.pallas_call(
        _node_mlp_body,
        grid=(rows // tile,),
        in_specs=in_specs,
        out_specs=pl.BlockSpec((tile, d_out), lambda i: (i, 0)),
        out_shape=jax.ShapeDtypeStruct((rows, d_out), jnp.float32),
    )(dyn2, agg2, *ws)


_CH = 128
_BPC = 16
_EB = _CH * _BPC


def _make_scatter(B, N, E_pad, F):
    NS = plsc.get_sparse_core_info().num_subcores
    NCH = E_pad // _CH
    NG = NCH // 8
    gper = NG // NS
    grem = NG - gper * NS
    NB = -(-(8 * (gper + 1)) // _BPC)
    mesh = plsc.VectorSubcoreMesh(core_axis_name="c", subcore_axis_name="s")

    @functools.partial(
        pl.kernel,
        out_type=jax.ShapeDtypeStruct((B * N, F), jnp.float32),
        mesh=mesh,
        scratch_types=[
            pltpu.VMEM((_BPC, _CH), jnp.int32),
            pltpu.VMEM((_EB, F), jnp.float32),
            pltpu.VMEM_SHARED((N, F), jnp.float32),
            pltpu.SemaphoreType.DMA,
        ],
        compiler_params=pltpu.CompilerParams(use_tc_tiling_on_sc=False),
    )
    def scatter_k(msg_hbm, recv_hbm, zeros_hbm, out_hbm, idx_v, msg_v, acc_sh, sem):
        b = lax.axis_index("c")
        s = lax.axis_index("s")

        @pl.when(s < NS - 1)
        def _():
            r0 = s * 640
            pltpu.sync_copy(zeros_hbm.at[pl.ds(r0, 640)], acc_sh.at[pl.ds(r0, 640)])

        @pl.when(s == NS - 1)
        def _():
            pltpu.sync_copy(zeros_hbm.at[pl.ds(9600, 400)], acc_sh.at[pl.ds(9600, 400)])

        plsc.subcore_barrier()

        base = 8 * (s * gper + jnp.minimum(s, grem))
        cnt = 8 * (gper + (s < grem).astype(jnp.int32))
        eoff = b * E_pad

        def blk_body(blk, carry):
            first = base + blk * _BPC
            start = jnp.minimum(first, base + cnt - _BPC)
            pltpu.sync_copy(recv_hbm.at[pl.ds(start, _BPC)], idx_v)
            pltpu.sync_copy(msg_hbm.at[pl.ds(eoff + start * _CH, _EB)], msg_v)
            for j in range(_BPC):
                @pl.when(start + j >= first)
                def _():
                    pltpu.async_copy(
                        msg_v.at[pl.ds(j * _CH, _CH)],
                        acc_sh.at[idx_v.at[j]], sem, add=True)
            for j in range(_BPC):
                @pl.when(start + j >= first)
                def _():
                    pltpu.make_async_copy(
                        msg_v.at[pl.ds(j * _CH, _CH)],
                        acc_sh.at[idx_v.at[j]], sem).wait()
            return carry

        lax.fori_loop(0, NB, blk_body, 0)
        plsc.subcore_barrier()

        @pl.when(s < NS - 1)
        def _():
            r0 = s * 640
            pltpu.sync_copy(acc_sh.at[pl.ds(r0, 640)],
                            out_hbm.at[pl.ds(b * N + r0, 640)])

        @pl.when(s == NS - 1)
        def _():
            pltpu.sync_copy(acc_sh.at[pl.ds(9600, 400)],
                            out_hbm.at[pl.ds(b * N + 9600, 400)])

    return scatter_k


def kernel(dyn_feats, rel_feats, senders, receivers, frel_params, fdyn_params):
    B, N, _ = dyn_feats.shape
    E = rel_feats.shape[1]
    F = frel_params[-1][0].shape[1]

    e_pad = -(-E // _ETILE) * _ETILE
    if (e_pad // _CH) % 8:
        e_pad = -(-(e_pad // _CH) // 8) * 8 * _CH

    msg = _edge_mlp(rel_feats, frel_params, e_pad)
    msg2 = msg.reshape(B * e_pad, F)

    recv_pad = jnp.zeros((e_pad,), jnp.int32).at[:E].set(receivers)
    recv_lin = recv_pad.reshape(-1, 8, _ETILE // 8).transpose(0, 2, 1).reshape(-1)
    recv2 = recv_lin.reshape(e_pad // _CH, _CH)
    zeros = jnp.zeros((N, F), jnp.float32)
    agg2 = _make_scatter(B, N, e_pad, F)(msg2, recv2, zeros)

    dyn2 = dyn_feats.reshape(B * N, dyn_feats.shape[-1])
    delta2 = _node_mlp(dyn2, agg2, fdyn_params, tile=5000)
    return delta2.reshape(B, N, fdyn_params[-1][0].shape[1])


# speedup vs baseline: 76.8868x; 1.0502x over previous
import functools

import jax
import jax.numpy as jnp
from jax import lax
from jax.experimental import pallas as pl
from jax.experimental.pallas import tpu as pltpu
from jax.experimental.pallas import tpu_sc as plsc


_ETILE = 16384


def _edge_mlp_body(relt, w1, b1, w2, b2, w3, b3, w4, b4, w5, b5, out):
    xt = relt[...].astype(jnp.bfloat16)
    def lyr(x, w, b):
        y = jnp.dot(x, w[...], preferred_element_type=jnp.float32) + b[...]
        return jnp.maximum(y, 0.0).astype(jnp.bfloat16)
    x = lax.dot_general(xt, w1[...], (((0,), (0,)), ((), ())),
                        preferred_element_type=jnp.float32) + b1[...]
    x = jnp.maximum(x, 0.0).astype(jnp.bfloat16)
    x = lyr(x, w2, b2)
    x = lyr(x, w3, b3)
    x = lyr(x, w4, b4)
    x = jnp.dot(x, w5[...], preferred_element_type=jnp.float32) + b5[...]
    q = _ETILE // 8
    for k in range(8):
        out[0, :, 16 * k:16 * (k + 1)] = x[q * k:q * (k + 1), :]


def _full(shape):
    return pl.BlockSpec(shape, lambda b, i: (0, 0))


def _edge_mlp(rel_feats, frel_params, e_pad):
    B, E, D = rel_feats.shape
    nb = e_pad // _ETILE
    relp = jnp.pad(rel_feats, ((0, 0), (0, e_pad - E), (0, 0)))
    relt = relp.transpose(2, 0, 1).reshape(D, B * e_pad)
    ws = []
    in_specs = [pl.BlockSpec((D, _ETILE), lambda b, i: (0, b * nb + i))]
    for (w, b) in frel_params:
        ws += [w.astype(jnp.bfloat16), b.reshape(1, -1)]
        in_specs += [_full(w.shape), _full((1, b.shape[0]))]
    f = frel_params[-1][0].shape[1]
    assert f == 16
    return pl.pallas_call(
        _edge_mlp_body,
        grid=(B, nb),
        in_specs=in_specs,
        out_specs=pl.BlockSpec((1, _ETILE // 8, 128), lambda b, i: (b, i, 0)),
        out_shape=jax.ShapeDtypeStruct((B, e_pad // 8, 128), jnp.float32),
    )(relt, *ws)


def _node_mlp_body(dyn, agg, w1a, w1b, b1, w2, b2, w3, b3, w4, b4, w5, b5, out):
    x = (jnp.dot(dyn[...], w1a[...], preferred_element_type=jnp.float32)
         + jnp.dot(agg[...], w1b[...], preferred_element_type=jnp.float32)
         + b1[...])
    x = jnp.maximum(x, 0.0)
    x = jnp.maximum(jnp.dot(x, w2[...], preferred_element_type=jnp.float32) + b2[...], 0.0)
    x = jnp.maximum(jnp.dot(x, w3[...], preferred_element_type=jnp.float32) + b3[...], 0.0)
    x = jnp.maximum(jnp.dot(x, w4[...], preferred_element_type=jnp.float32) + b4[...], 0.0)
    out[...] = jnp.dot(x, w5[...], preferred_element_type=jnp.float32) + b5[...]


def _nfull(shape):
    return pl.BlockSpec(shape, lambda i: (0, 0))


def _node_mlp(dyn2, agg2, fdyn_params, tile):
    rows = dyn2.shape[0]
    assert rows % tile == 0
    d_dyn = dyn2.shape[1]
    (w1, b1) = fdyn_params[0]
    ws = [w1[:d_dyn], w1[d_dyn:], b1.reshape(1, -1)]
    in_specs = [
        pl.BlockSpec((tile, d_dyn), lambda i: (i, 0)),
        pl.BlockSpec((tile, agg2.shape[1]), lambda i: (i, 0)),
        _nfull(ws[0].shape), _nfull(ws[1].shape), _nfull((1, b1.shape[0])),
    ]
    for (w, b) in fdyn_params[1:]:
        ws += [w, b.reshape(1, -1)]
        in_specs += [_nfull(w.shape), _nfull((1, b.shape[0]))]
    d_out = fdyn_params[-1][0].shape[1]
    return pl.pallas_call(
        _node_mlp_body,
        grid=(rows // tile,),
        in_specs=in_specs,
        out_specs=pl.BlockSpec((tile, d_out), lambda i: (i, 0)),
        out_shape=jax.ShapeDtypeStruct((rows, d_out), jnp.float32),
    )(dyn2, agg2, *ws)


_CH = 128
_BPC = 16
_EB = _CH * _BPC


def _make_scatter(B, N, E_pad, F):
    NS = plsc.get_sparse_core_info().num_subcores
    NCH = E_pad // _CH
    NG = NCH // 8
    gper = NG // NS
    grem = NG - gper * NS
    NB = -(-(8 * (gper + 1)) // _BPC)
    mesh = plsc.VectorSubcoreMesh(core_axis_name="c", subcore_axis_name="s")

    @functools.partial(
        pl.kernel,
        out_type=jax.ShapeDtypeStruct((B * N, F), jnp.float32),
        mesh=mesh,
        scratch_types=[
            pltpu.VMEM((_BPC, _CH), jnp.int32),
            pltpu.VMEM((_EB, F), jnp.float32),
            pltpu.VMEM_SHARED((N + 8, F), jnp.float32),
            pltpu.SemaphoreType.DMA,
        ],
        compiler_params=pltpu.CompilerParams(use_tc_tiling_on_sc=False),
    )
    def scatter_k(msg_hbm, recv_hbm, zeros_hbm, out_hbm, idx_v, msg_v, acc_sh, sem):
        b = lax.axis_index("c")
        s = lax.axis_index("s")

        @pl.when(s < NS - 1)
        def _():
            r0 = s * 640
            pltpu.sync_copy(zeros_hbm.at[pl.ds(r0, 640)], acc_sh.at[pl.ds(r0, 640)])

        @pl.when(s == NS - 1)
        def _():
            pltpu.sync_copy(zeros_hbm.at[pl.ds(9600, 400)], acc_sh.at[pl.ds(9600, 400)])

        plsc.subcore_barrier()

        base = 8 * (s * gper + jnp.minimum(s, grem))
        cnt = 8 * (gper + (s < grem).astype(jnp.int32))
        eoff = b * E_pad

        def blk_body(blk, carry):
            first = base + blk * _BPC
            start = jnp.minimum(first, base + cnt - _BPC)
            pltpu.sync_copy(recv_hbm.at[pl.ds(start, _BPC)], idx_v)
            pltpu.sync_copy(msg_hbm.at[pl.ds(eoff + start * _CH, _EB)], msg_v)
            for j in range(_BPC):
                @pl.when(start + j >= first)
                def _():
                    pltpu.async_copy(
                        msg_v.at[pl.ds(j * _CH, _CH)],
                        acc_sh.at[idx_v.at[j]], sem, add=True)
            for j in range(_BPC):
                @pl.when(start + j >= first)
                def _():
                    pltpu.make_async_copy(
                        msg_v.at[pl.ds(j * _CH, _CH)],
                        acc_sh.at[idx_v.at[j]], sem).wait()
            return carry

        lax.fori_loop(0, NB, blk_body, 0)
        plsc.subcore_barrier()

        @pl.when(s < NS - 1)
        def _():
            r0 = s * 640
            pltpu.sync_copy(acc_sh.at[pl.ds(r0, 640)],
                            out_hbm.at[pl.ds(b * N + r0, 640)])

        @pl.when(s == NS - 1)
        def _():
            pltpu.sync_copy(acc_sh.at[pl.ds(9600, 400)],
                            out_hbm.at[pl.ds(b * N + 9600, 400)])

    return scatter_k


def kernel(dyn_feats, rel_feats, senders, receivers, frel_params, fdyn_params):
    B, N, _ = dyn_feats.shape
    E = rel_feats.shape[1]
    F = frel_params[-1][0].shape[1]

    e_pad = -(-E // _ETILE) * _ETILE
    if (e_pad // _CH) % 8:
        e_pad = -(-(e_pad // _CH) // 8) * 8 * _CH

    msg = _edge_mlp(rel_feats, frel_params, e_pad)
    msg2 = msg.reshape(B * e_pad, F)

    trash = N + (jnp.arange(e_pad - E, dtype=jnp.int32) & 7)
    recv_pad = jnp.concatenate([receivers, trash])
    recv_lin = recv_pad.reshape(-1, 8, _ETILE // 8).transpose(0, 2, 1).reshape(-1)
    recv2 = recv_lin.reshape(e_pad // _CH, _CH)
    zeros = jnp.zeros((N, F), jnp.float32)
    agg2 = _make_scatter(B, N, e_pad, F)(msg2, recv2, zeros)

    dyn2 = dyn_feats.reshape(B * N, dyn_feats.shape[-1])
    delta2 = _node_mlp(dyn2, agg2, fdyn_params, tile=5000)
    return delta2.reshape(B, N, fdyn_params[-1][0].shape[1])
